# Initial kernel scaffold; baseline (speedup 1.0000x reference)
import jax
import jax.numpy as jnp
from jax.experimental import pallas as pl


def kernel(hidden_states, gate_w, c_fc_w, c_proj_w):
    raise NotImplementedError("write your pallas kernel here")


# speedup vs baseline: 2.0008x; 2.0008x over previous
import functools

import jax
import jax.numpy as jnp
from jax.experimental import pallas as pl

NUM_EXPERTS = 8
TOP_K = 2
HIDDEN = 1024
INTER = 1024
TOKENS = 2048


def _routing_kernel(x_ref, gw_ref, gates_ref):
    x = x_ref[:]
    logits = jax.lax.dot_general(
        x, gw_ref[:], (((1,), (1,)), ((), ())),
        preferred_element_type=jnp.float32)
    e_iota = jax.lax.broadcasted_iota(jnp.int32, logits.shape, 1)
    m1 = jnp.max(logits, axis=1, keepdims=True)
    idx1 = jnp.min(jnp.where(logits == m1, e_iota, NUM_EXPERTS),
                   axis=1, keepdims=True)
    oh1 = e_iota == idx1
    l2 = jnp.where(oh1, -jnp.inf, logits)
    m2 = jnp.max(l2, axis=1, keepdims=True)
    idx2 = jnp.min(jnp.where(l2 == m2, e_iota, NUM_EXPERTS),
                   axis=1, keepdims=True)
    oh2 = e_iota == idx2
    t = jnp.exp(m2 - m1)
    p1 = 1.0 / (1.0 + t)
    p2 = t / (1.0 + t)
    gates_ref[:] = jnp.where(oh1, p1, 0.0) + jnp.where(oh2, p2, 0.0)


def _expert_kernel(x_ref, fc_ref, proj_ref, gates_ref, out_ref):
    e = pl.program_id(0)
    x = x_ref[:]
    wfc = fc_ref[0]
    u = jax.lax.dot_general(x, wfc[:INTER], (((1,), (1,)), ((), ())),
                            preferred_element_type=jnp.float32)
    g = jax.lax.dot_general(x, wfc[INTER:], (((1,), (1,)), ((), ())),
                            preferred_element_type=jnp.float32)
    h = u * (g * jax.nn.sigmoid(g))
    y = jax.lax.dot_general(h, proj_ref[0], (((1,), (1,)), ((), ())),
                            preferred_element_type=jnp.float32)
    gates = gates_ref[:]
    e_iota = jax.lax.broadcasted_iota(jnp.int32, gates.shape, 1)
    col = jnp.sum(jnp.where(e_iota == e, gates, 0.0), axis=1, keepdims=True)
    contrib = col * y

    @pl.when(e == 0)
    def _():
        out_ref[:] = contrib

    @pl.when(e > 0)
    def _():
        out_ref[:] = out_ref[:] + contrib


@jax.jit
def kernel(hidden_states, gate_w, c_fc_w, c_proj_w):
    T, H = hidden_states.shape
    gates = pl.pallas_call(
        _routing_kernel,
        out_shape=jax.ShapeDtypeStruct((T, NUM_EXPERTS), jnp.float32),
    )(hidden_states, gate_w)

    out = pl.pallas_call(
        _expert_kernel,
        grid=(NUM_EXPERTS,),
        in_specs=[
            pl.BlockSpec((T, H), lambda e: (0, 0)),
            pl.BlockSpec((1, 2 * INTER, H), lambda e: (e, 0, 0)),
            pl.BlockSpec((1, H, INTER), lambda e: (e, 0, 0)),
            pl.BlockSpec((T, NUM_EXPERTS), lambda e: (0, 0)),
        ],
        out_specs=pl.BlockSpec((T, H), lambda e: (0, 0)),
        out_shape=jax.ShapeDtypeStruct((T, H), jnp.float32),
    )(hidden_states, c_fc_w, c_proj_w, gates)
    return out
